# baseline (speedup 1.0000x reference)
import functools

import jax
import jax.numpy as jnp
from jax import lax
from jax.experimental import pallas as pl
from jax.experimental.pallas import tpu as pltpu
from jax.experimental.pallas import tpu_sc as plsc

VOCAB = 32768
HIDDEN = 4096
NTOK = 2 * 8192

NC = 2
NS = 16
NW = NC * NS
PER_W = NTOK // NW
CH = 4
NCH = PER_W // CH
NBUF = 4
K = 2


def _embed(idx_hbm, table_hbm, out_hbm, idx_v, buf_v,
           g0, g1, g2, g3, w0, w1, w2, w3):
    gs = (g0, g1, g2, g3)
    ws = (w0, w1, w2, w3)
    wid = lax.axis_index("s") * NC + lax.axis_index("c")
    base = wid * PER_W
    pltpu.sync_copy(idx_hbm.at[wid], idx_v)
    for c in range(K):
        pltpu.async_copy(table_hbm.at[idx_v.at[c]], buf_v.at[c], gs[c])

    def group(i, carry):
        g = i * NBUF
        for b in range(NBUF):
            c = g + b
            pltpu.make_async_copy(
                table_hbm.at[idx_v.at[c]], buf_v.at[b], gs[b]).wait()
            pltpu.async_copy(
                buf_v.at[b], out_hbm.at[pl.ds(base + c * CH, CH)], ws[b])
            b2 = (b + K) % NBUF

            @pl.when(c + K < NCH)
            def _():
                @pl.when(c + K >= NBUF)
                def _():
                    pltpu.make_async_copy(
                        buf_v.at[b2],
                        out_hbm.at[pl.ds(base + (c + K - NBUF) * CH, CH)],
                        ws[b2]).wait()
                pltpu.async_copy(
                    table_hbm.at[idx_v.at[c + K]], buf_v.at[b2], gs[b2])
        return carry

    lax.fori_loop(0, NCH // NBUF, group, 0)
    for b in range(NBUF):
        c = NCH - NBUF + b
        pltpu.make_async_copy(
            buf_v.at[b], out_hbm.at[pl.ds(base + c * CH, CH)], ws[b]).wait()


@jax.jit
def kernel(tokens, table):
    idx = tokens.astype(jnp.int32).reshape(NW, NCH, CH)
    mesh = plsc.VectorSubcoreMesh(core_axis_name="c", subcore_axis_name="s")
    emb = functools.partial(
        pl.kernel,
        mesh=mesh,
        out_type=jax.ShapeDtypeStruct((NTOK, HIDDEN), jnp.float32),
        scratch_types=[
            pltpu.VMEM((NCH, CH), jnp.int32),
            pltpu.VMEM((NBUF, CH, HIDDEN), jnp.float32),
        ] + [pltpu.SemaphoreType.DMA] * (2 * NBUF),
    )(_embed)
    out = emb(idx, table)
    return out.reshape(2, 8192, HIDDEN)
